# Initial kernel scaffold; baseline (speedup 1.0000x reference)
import jax
import jax.numpy as jnp
from jax.experimental import pallas as pl


def kernel(pretrained, finetuned):
    raise NotImplementedError("write your pallas kernel here")


# speedup vs baseline: 10.8037x; 10.8037x over previous
import functools

import jax
import jax.numpy as jnp
from jax import lax
from jax.experimental import pallas as pl
from jax.experimental.pallas import tpu as pltpu
from jax.experimental.pallas import tpu_sc as plsc

N = 16777216
SPARSITY = 0.01
SIGMOID_BIAS = 5.0
K = int(SPARSITY * N)

NC, NS, LANES = 2, 16, 16
NTILES = NC * NS
PER_TILE = N // NTILES
BLK = 2048
NBLK = PER_TILE // BLK
VPB = BLK // LANES

B1, B2, B3 = 4096, 4096, 128

_mesh = plsc.VectorSubcoreMesh(core_axis_name="c", subcore_axis_name="s")
_sc_params = pltpu.CompilerParams(needs_layout_passes=False)


def _wid():
    return lax.axis_index("s") * NC + lax.axis_index("c")


def _zero_hist(hist_v, nbuckets):
    zeros = jnp.zeros((LANES,), jnp.int32)

    def body(i, _):
        hist_v[pl.ds(i * LANES, LANES)] = zeros
        return 0

    lax.fori_loop(0, (LANES * nbuckets) // LANES, body, 0)


@functools.partial(
    pl.kernel,
    out_type=[
        jax.ShapeDtypeStruct((N,), jnp.float32),
        jax.ShapeDtypeStruct((NTILES, LANES * B1), jnp.int32),
    ],
    mesh=_mesh,
    scratch_types=[
        pltpu.VMEM((BLK,), jnp.float32),
        pltpu.VMEM((BLK,), jnp.float32),
        pltpu.VMEM((BLK,), jnp.float32),
        pltpu.VMEM((LANES * B1,), jnp.int32),
    ],
    compiler_params=_sc_params,
)
def _p1(p_hbm, f_hbm, tv_hbm, hist_hbm, p_v, f_v, tv_v, hist_v):
    wid = _wid()
    base = wid * PER_TILE
    _zero_hist(hist_v, B1)
    laneoff = lax.iota(jnp.int32, LANES) * B1
    ones = jnp.ones((LANES,), jnp.int32)

    def blk(b, _):
        off = base + b * BLK
        pltpu.sync_copy(p_hbm.at[pl.ds(off, BLK)], p_v)
        pltpu.sync_copy(f_hbm.at[pl.ds(off, BLK)], f_v)

        def vec(i, _):
            s = i * LANES
            t = f_v[pl.ds(s, LANES)] - p_v[pl.ds(s, LANES)]
            tv_v[pl.ds(s, LANES)] = t
            bits = plsc.bitcast(jnp.abs(t), jnp.int32)
            idx = laneoff + lax.shift_right_logical(bits, 19)
            plsc.addupdate_scatter(hist_v, [idx], ones)
            return 0

        lax.fori_loop(0, VPB, vec, 0)
        pltpu.sync_copy(tv_v, tv_hbm.at[pl.ds(off, BLK)])
        return 0

    lax.fori_loop(0, NBLK, blk, 0)
    pltpu.sync_copy(hist_v, hist_hbm.at[wid])


def _make_masked_hist(shift_match, shift_bucket, bmask, nbuckets):
    @functools.partial(
        pl.kernel,
        out_type=jax.ShapeDtypeStruct((NTILES, LANES * nbuckets), jnp.int32),
        mesh=_mesh,
        scratch_types=[
            pltpu.VMEM((BLK,), jnp.float32),
            pltpu.VMEM((LANES,), jnp.int32),
            pltpu.VMEM((LANES * nbuckets,), jnp.int32),
        ],
        compiler_params=_sc_params,
    )
    def hist_pass(tv_hbm, sel_hbm, hist_hbm, tv_v, sel_v, hist_v):
        wid = _wid()
        base = wid * PER_TILE
        _zero_hist(hist_v, nbuckets)
        pltpu.sync_copy(sel_hbm, sel_v)
        sel = sel_v[...]
        laneoff = lax.iota(jnp.int32, LANES) * nbuckets
        ones = jnp.ones((LANES,), jnp.int32)
        zeros = jnp.zeros((LANES,), jnp.int32)

        def blk(b, _):
            off = base + b * BLK
            pltpu.sync_copy(tv_hbm.at[pl.ds(off, BLK)], tv_v)

            def vec(i, _):
                t = tv_v[pl.ds(i * LANES, LANES)]
                bits = plsc.bitcast(jnp.abs(t), jnp.int32)
                match = lax.shift_right_logical(bits, shift_match) == sel
                bucket = jnp.bitwise_and(
                    lax.shift_right_logical(bits, shift_bucket), bmask)
                idx = laneoff + bucket
                plsc.addupdate_scatter(hist_v, [idx], jnp.where(match, ones, zeros))
                return 0

            lax.fori_loop(0, VPB, vec, 0)
            return 0

        lax.fori_loop(0, NBLK, blk, 0)
        pltpu.sync_copy(hist_v, hist_hbm.at[wid])

    return hist_pass


_p2 = _make_masked_hist(19, 7, B2 - 1, B2)
_p3 = _make_masked_hist(7, 0, B3 - 1, B3)


def _select(hist, rank, nbuckets, nbits):
    rows = hist.size // nbuckets
    h2 = hist.reshape(rows, nbuckets)

    def body(h_ref, r_ref, sel_ref, rn_ref):
        h = h_ref[...]
        col = lax.broadcasted_iota(jnp.int32, h.shape, 1)
        k = r_ref[0, 0]

        def step(_, lohi):
            lo, hi = lohi
            mid = (lo + hi) // 2
            s = jnp.sum(jnp.where(col >= mid, h, 0))
            ok = s >= k
            return jnp.where(ok, mid, lo), jnp.where(ok, hi, mid)

        lo, _hi = lax.fori_loop(
            0, nbits, step, (jnp.int32(0), jnp.int32(nbuckets)))
        above = jnp.sum(jnp.where(col >= lo + 1, h, 0))
        sel_ref[0, 0] = lo
        rn_ref[0, 0] = k - above

    return pl.pallas_call(
        body,
        in_specs=[
            pl.BlockSpec(memory_space=pltpu.VMEM),
            pl.BlockSpec(memory_space=pltpu.SMEM),
        ],
        out_specs=[
            pl.BlockSpec(memory_space=pltpu.SMEM),
            pl.BlockSpec(memory_space=pltpu.SMEM),
        ],
        out_shape=[
            jax.ShapeDtypeStruct((1, 1), jnp.int32),
            jax.ShapeDtypeStruct((1, 1), jnp.int32),
        ],
    )(h2, rank)


_SIG_HI = 0.9933071490757153
_SIG_LO = 0.0066928509242848554

_EW_COLS = 1024
_EW_ROWS = N // _EW_COLS
_EW_GRID = 32
_EW_RB = _EW_ROWS // _EW_GRID


def _ew_body(thr_ref, tv_ref, mask_ref, delta_ref):
    thr = thr_ref[0, 0]
    t = tv_ref[...]
    above = jnp.abs(t) > thr
    mask_ref[...] = jnp.where(above, SIGMOID_BIAS, -SIGMOID_BIAS).astype(jnp.float32)
    delta_ref[...] = t * jnp.where(above, _SIG_HI, _SIG_LO).astype(jnp.float32)


def _elementwise(tv2d, thr):
    return pl.pallas_call(
        _ew_body,
        grid=(_EW_GRID,),
        in_specs=[
            pl.BlockSpec(memory_space=pltpu.SMEM),
            pl.BlockSpec((_EW_RB, _EW_COLS), lambda i: (i, 0)),
        ],
        out_specs=[
            pl.BlockSpec((_EW_RB, _EW_COLS), lambda i: (i, 0)),
            pl.BlockSpec((_EW_RB, _EW_COLS), lambda i: (i, 0)),
        ],
        out_shape=[
            jax.ShapeDtypeStruct((_EW_ROWS, _EW_COLS), jnp.float32),
            jax.ShapeDtypeStruct((_EW_ROWS, _EW_COLS), jnp.float32),
        ],
    )(thr, tv2d)


def kernel(pretrained, finetuned):
    tv, h1 = _p1(pretrained, finetuned)

    k0 = jnp.full((1, 1), K, jnp.int32)
    j1, r2 = _select(h1, k0, B1, 12)
    sel1 = jnp.full((LANES,), j1[0, 0], jnp.int32)

    h2 = _p2(tv, sel1)
    j2, r3 = _select(h2, r2, B2, 12)
    sel12 = jnp.full((LANES,), (j1[0, 0] << 12) | j2[0, 0], jnp.int32)

    h3 = _p3(tv, sel12)
    j3, r4 = _select(h3, r3, B3, 7)

    bits = (j1[0, 0] << 19) | (j2[0, 0] << 7) | j3[0, 0]
    thr = lax.bitcast_convert_type(bits, jnp.float32)

    mask2d, delta2d = _elementwise(tv.reshape(_EW_ROWS, _EW_COLS),
                                   jnp.full((1, 1), thr, jnp.float32))

    cnt_above = K - r4[0, 0]
    prop = (5 * (2 * cnt_above - N)).astype(jnp.float32) / jnp.float32(N)

    return (delta2d.reshape(N), mask2d.reshape(N), prop)


# speedup vs baseline: 15.0718x; 1.3951x over previous
import functools

import jax
import jax.numpy as jnp
from jax import lax
from jax.experimental import pallas as pl
from jax.experimental.pallas import tpu as pltpu
from jax.experimental.pallas import tpu_sc as plsc

N = 16777216
SPARSITY = 0.01
SIGMOID_BIAS = 5.0
K = int(SPARSITY * N)

NC, NS, LANES = 2, 16, 16
NTILES = NC * NS
PER_TILE = N // NTILES
NBUF = 2
UNROLL = 4

B1, B2, B3 = 4096, 4096, 128

_mesh = plsc.VectorSubcoreMesh(core_axis_name="c", subcore_axis_name="s")
_sc_params = pltpu.CompilerParams(needs_layout_passes=False)


def _wid():
    return lax.axis_index("s") * NC + lax.axis_index("c")


def _zero_hist(hist_v, nwords):
    zeros = jnp.zeros((LANES,), jnp.int32)

    def body(i, _):
        for u in range(UNROLL):
            hist_v[pl.ds((i * UNROLL + u) * LANES, LANES)] = zeros
        return 0

    lax.fori_loop(0, nwords // (LANES * UNROLL), body, 0)


_P1_BLK = 8192
_P1_NBLK = PER_TILE // _P1_BLK


@functools.partial(
    pl.kernel,
    out_type=[
        jax.ShapeDtypeStruct((N,), jnp.float32),
        jax.ShapeDtypeStruct((NTILES, LANES * B1), jnp.int32),
    ],
    mesh=_mesh,
    scratch_types=[
        pltpu.VMEM((NBUF, _P1_BLK), jnp.float32),
        pltpu.VMEM((NBUF, _P1_BLK), jnp.float32),
        pltpu.VMEM((NBUF, _P1_BLK), jnp.float32),
        pltpu.VMEM((LANES * B1,), jnp.int32),
        pltpu.SemaphoreType.DMA,
        pltpu.SemaphoreType.DMA,
        pltpu.SemaphoreType.DMA,
        pltpu.SemaphoreType.DMA,
        pltpu.SemaphoreType.DMA,
        pltpu.SemaphoreType.DMA,
    ],
    compiler_params=_sc_params,
)
def _p1(p_hbm, f_hbm, tv_hbm, hist_hbm, p_v, f_v, t_v, hist_v,
        sp0, sp1, sf0, sf1, ss0, ss1):
    wid = _wid()
    base = wid * PER_TILE
    sp = (sp0, sp1)
    sf = (sf0, sf1)
    ss = (ss0, ss1)
    for j in range(NBUF):
        off = base + j * _P1_BLK
        pltpu.async_copy(p_hbm.at[pl.ds(off, _P1_BLK)], p_v.at[j], sp[j])
        pltpu.async_copy(f_hbm.at[pl.ds(off, _P1_BLK)], f_v.at[j], sf[j])
    _zero_hist(hist_v, LANES * B1)
    laneoff = lax.iota(jnp.int32, LANES) * B1
    ones = jnp.ones((LANES,), jnp.int32)

    def outer(g, _):
        for j in range(NBUF):
            blk = g * NBUF + j
            off = base + blk * _P1_BLK
            pltpu.make_async_copy(
                p_hbm.at[pl.ds(off, _P1_BLK)], p_v.at[j], sp[j]).wait()
            pltpu.make_async_copy(
                f_hbm.at[pl.ds(off, _P1_BLK)], f_v.at[j], sf[j]).wait()

            @pl.when(blk >= NBUF)
            def _wait_store():
                pltpu.make_async_copy(
                    t_v.at[j], tv_hbm.at[pl.ds(base, _P1_BLK)], ss[j]).wait()

            def vec(i, _):
                for u in range(UNROLL):
                    s = (i * UNROLL + u) * LANES
                    t = f_v[j, pl.ds(s, LANES)] - p_v[j, pl.ds(s, LANES)]
                    t_v[j, pl.ds(s, LANES)] = t
                    bits = plsc.bitcast(jnp.abs(t), jnp.int32)
                    idx = laneoff + lax.shift_right_logical(bits, 19)
                    plsc.addupdate_scatter(hist_v, [idx], ones)
                return 0

            lax.fori_loop(0, _P1_BLK // (LANES * UNROLL), vec, 0)
            pltpu.async_copy(t_v.at[j], tv_hbm.at[pl.ds(off, _P1_BLK)], ss[j])

            @pl.when(blk + NBUF < _P1_NBLK)
            def _next_load():
                noff = base + (blk + NBUF) * _P1_BLK
                pltpu.async_copy(
                    p_hbm.at[pl.ds(noff, _P1_BLK)], p_v.at[j], sp[j])
                pltpu.async_copy(
                    f_hbm.at[pl.ds(noff, _P1_BLK)], f_v.at[j], sf[j])
        return 0

    lax.fori_loop(0, _P1_NBLK // NBUF, outer, 0)
    for j in range(NBUF):
        pltpu.make_async_copy(
            t_v.at[j], tv_hbm.at[pl.ds(base, _P1_BLK)], ss[j]).wait()
    pltpu.sync_copy(hist_v, hist_hbm.at[wid])


def _make_masked_hist(shift_match, shift_bucket, bmask, nbuckets, blk):
    nblk = PER_TILE // blk

    @functools.partial(
        pl.kernel,
        out_type=jax.ShapeDtypeStruct((NTILES, LANES * nbuckets), jnp.int32),
        mesh=_mesh,
        scratch_types=[
            pltpu.VMEM((NBUF, blk), jnp.float32),
            pltpu.VMEM((LANES,), jnp.int32),
            pltpu.VMEM((LANES * nbuckets,), jnp.int32),
            pltpu.SemaphoreType.DMA,
            pltpu.SemaphoreType.DMA,
        ],
        compiler_params=_sc_params,
    )
    def hist_pass(tv_hbm, sel_hbm, hist_hbm, tv_v, sel_v, hist_v, s0, s1):
        wid = _wid()
        base = wid * PER_TILE
        sems = (s0, s1)
        for j in range(NBUF):
            off = base + j * blk
            pltpu.async_copy(tv_hbm.at[pl.ds(off, blk)], tv_v.at[j], sems[j])
        pltpu.sync_copy(sel_hbm, sel_v)
        _zero_hist(hist_v, LANES * nbuckets)
        sel = sel_v[...]
        laneoff = lax.iota(jnp.int32, LANES) * nbuckets
        ones = jnp.ones((LANES,), jnp.int32)
        zeros = jnp.zeros((LANES,), jnp.int32)

        def outer(g, _):
            for j in range(NBUF):
                b = g * NBUF + j
                off = base + b * blk
                pltpu.make_async_copy(
                    tv_hbm.at[pl.ds(off, blk)], tv_v.at[j], sems[j]).wait()

                def vec(i, _):
                    for u in range(UNROLL):
                        s = (i * UNROLL + u) * LANES
                        t = tv_v[j, pl.ds(s, LANES)]
                        bits = plsc.bitcast(jnp.abs(t), jnp.int32)
                        match = lax.shift_right_logical(bits, shift_match) == sel
                        bucket = jnp.bitwise_and(
                            lax.shift_right_logical(bits, shift_bucket), bmask)
                        idx = laneoff + bucket
                        plsc.addupdate_scatter(
                            hist_v, [idx], jnp.where(match, ones, zeros))
                    return 0

                lax.fori_loop(0, blk // (LANES * UNROLL), vec, 0)

                @pl.when(b + NBUF < nblk)
                def _next_load():
                    noff = base + (b + NBUF) * blk
                    pltpu.async_copy(
                        tv_hbm.at[pl.ds(noff, blk)], tv_v.at[j], sems[j])
            return 0

        lax.fori_loop(0, nblk // NBUF, outer, 0)
        pltpu.sync_copy(hist_v, hist_hbm.at[wid])

    return hist_pass


_p2 = _make_masked_hist(19, 7, B2 - 1, B2, 16384)
_p3 = _make_masked_hist(7, 0, B3 - 1, B3, 32768)


def _select(hist, rank, nbuckets, nbits):
    rows = hist.size // nbuckets
    h2 = hist.reshape(rows, nbuckets)

    def body(h_ref, r_ref, sel_ref, rn_ref):
        h = h_ref[...]
        col = lax.broadcasted_iota(jnp.int32, h.shape, 1)
        k = r_ref[0, 0]

        def step(_, lohi):
            lo, hi = lohi
            mid = (lo + hi) // 2
            s = jnp.sum(jnp.where(col >= mid, h, 0))
            ok = s >= k
            return jnp.where(ok, mid, lo), jnp.where(ok, hi, mid)

        lo, _hi = lax.fori_loop(
            0, nbits, step, (jnp.int32(0), jnp.int32(nbuckets)))
        above = jnp.sum(jnp.where(col >= lo + 1, h, 0))
        sel_ref[0, 0] = lo
        rn_ref[0, 0] = k - above

    return pl.pallas_call(
        body,
        in_specs=[
            pl.BlockSpec(memory_space=pltpu.VMEM),
            pl.BlockSpec(memory_space=pltpu.SMEM),
        ],
        out_specs=[
            pl.BlockSpec(memory_space=pltpu.SMEM),
            pl.BlockSpec(memory_space=pltpu.SMEM),
        ],
        out_shape=[
            jax.ShapeDtypeStruct((1, 1), jnp.int32),
            jax.ShapeDtypeStruct((1, 1), jnp.int32),
        ],
    )(h2, rank)


_SIG_HI = 0.9933071490757153
_SIG_LO = 0.0066928509242848554

_EW_COLS = 1024
_EW_ROWS = N // _EW_COLS
_EW_GRID = 32
_EW_RB = _EW_ROWS // _EW_GRID


def _ew_body(thr_ref, tv_ref, mask_ref, delta_ref):
    thr = thr_ref[0, 0]
    t = tv_ref[...]
    above = jnp.abs(t) > thr
    mask_ref[...] = jnp.where(above, SIGMOID_BIAS, -SIGMOID_BIAS).astype(jnp.float32)
    delta_ref[...] = t * jnp.where(above, _SIG_HI, _SIG_LO).astype(jnp.float32)


def _elementwise(tv2d, thr):
    return pl.pallas_call(
        _ew_body,
        grid=(_EW_GRID,),
        in_specs=[
            pl.BlockSpec(memory_space=pltpu.SMEM),
            pl.BlockSpec((_EW_RB, _EW_COLS), lambda i: (i, 0)),
        ],
        out_specs=[
            pl.BlockSpec((_EW_RB, _EW_COLS), lambda i: (i, 0)),
            pl.BlockSpec((_EW_RB, _EW_COLS), lambda i: (i, 0)),
        ],
        out_shape=[
            jax.ShapeDtypeStruct((_EW_ROWS, _EW_COLS), jnp.float32),
            jax.ShapeDtypeStruct((_EW_ROWS, _EW_COLS), jnp.float32),
        ],
    )(thr, tv2d)


def kernel(pretrained, finetuned):
    tv, h1 = _p1(pretrained, finetuned)

    k0 = jnp.full((1, 1), K, jnp.int32)
    j1, r2 = _select(h1, k0, B1, 12)
    sel1 = jnp.full((LANES,), j1[0, 0], jnp.int32)

    h2 = _p2(tv, sel1)
    j2, r3 = _select(h2, r2, B2, 12)
    sel12 = jnp.full((LANES,), (j1[0, 0] << 12) | j2[0, 0], jnp.int32)

    h3 = _p3(tv, sel12)
    j3, r4 = _select(h3, r3, B3, 7)

    bits = (j1[0, 0] << 19) | (j2[0, 0] << 7) | j3[0, 0]
    thr = lax.bitcast_convert_type(bits, jnp.float32)

    mask2d, delta2d = _elementwise(tv.reshape(_EW_ROWS, _EW_COLS),
                                   jnp.full((1, 1), thr, jnp.float32))

    cnt_above = K - r4[0, 0]
    prop = (5 * (2 * cnt_above - N)).astype(jnp.float32) / jnp.float32(N)

    return (delta2d.reshape(N), mask2d.reshape(N), prop)


# speedup vs baseline: 19.5504x; 1.2971x over previous
import functools

import jax
import jax.numpy as jnp
from jax import lax
from jax.experimental import pallas as pl
from jax.experimental.pallas import tpu as pltpu
from jax.experimental.pallas import tpu_sc as plsc

N = 16777216
SPARSITY = 0.01
SIGMOID_BIAS = 5.0
K = int(SPARSITY * N)

NC, NS, LANES = 2, 16, 16
NTILES = NC * NS
PER_TILE = N // NTILES
NBUF = 2
UNROLL = 8

B1, B2, B3 = 4096, 4096, 128

_mesh = plsc.VectorSubcoreMesh(core_axis_name="c", subcore_axis_name="s")
_sc_params = pltpu.CompilerParams(needs_layout_passes=False)


def _wid():
    return lax.axis_index("s") * NC + lax.axis_index("c")


def _zero_hist(hist_v, nwords):
    zeros = jnp.zeros((LANES,), jnp.int32)

    def body(i, _):
        for u in range(UNROLL):
            hist_v[pl.ds((i * UNROLL + u) * LANES, LANES)] = zeros
        return 0

    lax.fori_loop(0, nwords // (LANES * UNROLL), body, 0)


_P1_BLK = 8192
_P1_NBLK = PER_TILE // _P1_BLK


@functools.partial(
    pl.kernel,
    out_type=[
        jax.ShapeDtypeStruct((N,), jnp.float32),
        jax.ShapeDtypeStruct((NTILES, LANES * B1), jnp.int32),
    ],
    mesh=_mesh,
    scratch_types=[
        pltpu.VMEM((NBUF, _P1_BLK), jnp.float32),
        pltpu.VMEM((NBUF, _P1_BLK), jnp.float32),
        pltpu.VMEM((NBUF, _P1_BLK), jnp.float32),
        pltpu.VMEM((LANES * B1,), jnp.int32),
        pltpu.SemaphoreType.DMA,
        pltpu.SemaphoreType.DMA,
        pltpu.SemaphoreType.DMA,
        pltpu.SemaphoreType.DMA,
        pltpu.SemaphoreType.DMA,
        pltpu.SemaphoreType.DMA,
    ],
    compiler_params=_sc_params,
)
def _p1(p_hbm, f_hbm, tv_hbm, hist_hbm, p_v, f_v, t_v, hist_v,
        sp0, sp1, sf0, sf1, ss0, ss1):
    wid = _wid()
    base = wid * PER_TILE
    sp = (sp0, sp1)
    sf = (sf0, sf1)
    ss = (ss0, ss1)
    for j in range(NBUF):
        off = base + j * _P1_BLK
        pltpu.async_copy(p_hbm.at[pl.ds(off, _P1_BLK)], p_v.at[j], sp[j])
        pltpu.async_copy(f_hbm.at[pl.ds(off, _P1_BLK)], f_v.at[j], sf[j])
    _zero_hist(hist_v, LANES * B1)
    lane = lax.iota(jnp.int32, LANES)
    ones = jnp.ones((LANES,), jnp.int32)

    def outer(g, _):
        for j in range(NBUF):
            blk = g * NBUF + j
            off = base + blk * _P1_BLK
            pltpu.make_async_copy(
                p_hbm.at[pl.ds(off, _P1_BLK)], p_v.at[j], sp[j]).wait()
            pltpu.make_async_copy(
                f_hbm.at[pl.ds(off, _P1_BLK)], f_v.at[j], sf[j]).wait()

            @pl.when(blk >= NBUF)
            def _wait_store():
                pltpu.make_async_copy(
                    t_v.at[j], tv_hbm.at[pl.ds(base, _P1_BLK)], ss[j]).wait()

            def vec(i, _):
                for u in range(UNROLL):
                    s = (i * UNROLL + u) * LANES
                    t = f_v[j, pl.ds(s, LANES)] - p_v[j, pl.ds(s, LANES)]
                    t_v[j, pl.ds(s, LANES)] = t
                    bits = plsc.bitcast(jnp.abs(t), jnp.int32)
                    bkt = lax.shift_right_logical(bits, 19)
                    idx = lax.shift_left(bkt, 4) + lane
                    plsc.addupdate_scatter(hist_v, [idx], ones)
                return 0

            lax.fori_loop(0, _P1_BLK // (LANES * UNROLL), vec, 0)
            pltpu.async_copy(t_v.at[j], tv_hbm.at[pl.ds(off, _P1_BLK)], ss[j])

            @pl.when(blk + NBUF < _P1_NBLK)
            def _next_load():
                noff = base + (blk + NBUF) * _P1_BLK
                pltpu.async_copy(
                    p_hbm.at[pl.ds(noff, _P1_BLK)], p_v.at[j], sp[j])
                pltpu.async_copy(
                    f_hbm.at[pl.ds(noff, _P1_BLK)], f_v.at[j], sf[j])
        return 0

    lax.fori_loop(0, _P1_NBLK // NBUF, outer, 0)
    for j in range(NBUF):
        pltpu.make_async_copy(
            t_v.at[j], tv_hbm.at[pl.ds(base, _P1_BLK)], ss[j]).wait()
    pltpu.sync_copy(hist_v, hist_hbm.at[wid])


def _make_masked_hist(shift_match, shift_bucket, bmask, nbuckets, blk):
    nblk = PER_TILE // blk

    @functools.partial(
        pl.kernel,
        out_type=jax.ShapeDtypeStruct((NTILES, LANES * nbuckets), jnp.int32),
        mesh=_mesh,
        scratch_types=[
            pltpu.VMEM((NBUF, blk), jnp.float32),
            pltpu.VMEM((LANES,), jnp.int32),
            pltpu.VMEM((LANES * nbuckets,), jnp.int32),
            pltpu.SemaphoreType.DMA,
            pltpu.SemaphoreType.DMA,
        ],
        compiler_params=_sc_params,
    )
    def hist_pass(tv_hbm, sel_hbm, hist_hbm, tv_v, sel_v, hist_v, s0, s1):
        wid = _wid()
        base = wid * PER_TILE
        sems = (s0, s1)
        for j in range(NBUF):
            off = base + j * blk
            pltpu.async_copy(tv_hbm.at[pl.ds(off, blk)], tv_v.at[j], sems[j])
        pltpu.sync_copy(sel_hbm, sel_v)
        _zero_hist(hist_v, LANES * nbuckets)
        sel = sel_v[...]
        lane = lax.iota(jnp.int32, LANES)
        ones = jnp.ones((LANES,), jnp.int32)
        zeros = jnp.zeros((LANES,), jnp.int32)

        def outer(g, _):
            for j in range(NBUF):
                b = g * NBUF + j
                off = base + b * blk
                pltpu.make_async_copy(
                    tv_hbm.at[pl.ds(off, blk)], tv_v.at[j], sems[j]).wait()

                def vec(i, _):
                    for u in range(UNROLL):
                        s = (i * UNROLL + u) * LANES
                        t = tv_v[j, pl.ds(s, LANES)]
                        bits = plsc.bitcast(jnp.abs(t), jnp.int32)
                        match = lax.shift_right_logical(bits, shift_match) == sel
                        bucket = jnp.bitwise_and(
                            lax.shift_right_logical(bits, shift_bucket), bmask)
                        idx = lax.shift_left(bucket, 4) + lane
                        plsc.addupdate_scatter(
                            hist_v, [idx], jnp.where(match, ones, zeros))
                    return 0

                lax.fori_loop(0, blk // (LANES * UNROLL), vec, 0)

                @pl.when(b + NBUF < nblk)
                def _next_load():
                    noff = base + (b + NBUF) * blk
                    pltpu.async_copy(
                        tv_hbm.at[pl.ds(noff, blk)], tv_v.at[j], sems[j])
            return 0

        lax.fori_loop(0, nblk // NBUF, outer, 0)
        pltpu.sync_copy(hist_v, hist_hbm.at[wid])

    return hist_pass


_p2 = _make_masked_hist(19, 7, B2 - 1, B2, 16384)
_p3 = _make_masked_hist(7, 0, B3 - 1, B3, 32768)


def _select(hist, rank, nbuckets, nbits):
    def body(h_ref, r_ref, sel_ref, rn_ref):
        h = jnp.sum(h_ref[...], axis=0, keepdims=True)
        col = lax.shift_right_logical(
            lax.broadcasted_iota(jnp.int32, h.shape, 1), 4)
        k = r_ref[0, 0]

        def step(_, lohi):
            lo, hi = lohi
            mid = (lo + hi) // 2
            s = jnp.sum(jnp.where(col >= mid, h, 0))
            ok = s >= k
            return jnp.where(ok, mid, lo), jnp.where(ok, hi, mid)

        lo, _hi = lax.fori_loop(
            0, nbits, step, (jnp.int32(0), jnp.int32(nbuckets)))
        above = jnp.sum(jnp.where(col >= lo + 1, h, 0))
        sel_ref[0, 0] = lo
        rn_ref[0, 0] = k - above

    return pl.pallas_call(
        body,
        in_specs=[
            pl.BlockSpec(memory_space=pltpu.VMEM),
            pl.BlockSpec(memory_space=pltpu.SMEM),
        ],
        out_specs=[
            pl.BlockSpec(memory_space=pltpu.SMEM),
            pl.BlockSpec(memory_space=pltpu.SMEM),
        ],
        out_shape=[
            jax.ShapeDtypeStruct((1, 1), jnp.int32),
            jax.ShapeDtypeStruct((1, 1), jnp.int32),
        ],
    )(hist, rank)


_SIG_HI = 0.9933071490757153
_SIG_LO = 0.0066928509242848554

_EW_COLS = 1024
_EW_ROWS = N // _EW_COLS
_EW_GRID = 32
_EW_RB = _EW_ROWS // _EW_GRID


def _ew_body(thr_ref, tv_ref, mask_ref, delta_ref):
    thr = thr_ref[0, 0]
    t = tv_ref[...]
    above = jnp.abs(t) > thr
    mask_ref[...] = jnp.where(above, SIGMOID_BIAS, -SIGMOID_BIAS).astype(jnp.float32)
    delta_ref[...] = t * jnp.where(above, _SIG_HI, _SIG_LO).astype(jnp.float32)


def _elementwise(tv2d, thr):
    return pl.pallas_call(
        _ew_body,
        grid=(_EW_GRID,),
        in_specs=[
            pl.BlockSpec(memory_space=pltpu.SMEM),
            pl.BlockSpec((_EW_RB, _EW_COLS), lambda i: (i, 0)),
        ],
        out_specs=[
            pl.BlockSpec((_EW_RB, _EW_COLS), lambda i: (i, 0)),
            pl.BlockSpec((_EW_RB, _EW_COLS), lambda i: (i, 0)),
        ],
        out_shape=[
            jax.ShapeDtypeStruct((_EW_ROWS, _EW_COLS), jnp.float32),
            jax.ShapeDtypeStruct((_EW_ROWS, _EW_COLS), jnp.float32),
        ],
    )(thr, tv2d)


def kernel(pretrained, finetuned):
    tv, h1 = _p1(pretrained, finetuned)

    k0 = jnp.full((1, 1), K, jnp.int32)
    j1, r2 = _select(h1, k0, B1, 12)
    sel1 = jnp.full((LANES,), j1[0, 0], jnp.int32)

    h2 = _p2(tv, sel1)
    j2, r3 = _select(h2, r2, B2, 12)
    sel12 = jnp.full((LANES,), (j1[0, 0] << 12) | j2[0, 0], jnp.int32)

    h3 = _p3(tv, sel12)
    j3, r4 = _select(h3, r3, B3, 7)

    bits = (j1[0, 0] << 19) | (j2[0, 0] << 7) | j3[0, 0]
    thr = lax.bitcast_convert_type(bits, jnp.float32)

    mask2d, delta2d = _elementwise(tv.reshape(_EW_ROWS, _EW_COLS),
                                   jnp.full((1, 1), thr, jnp.float32))

    cnt_above = K - r4[0, 0]
    prop = (5 * (2 * cnt_above - N)).astype(jnp.float32) / jnp.float32(N)

    return (delta2d.reshape(N), mask2d.reshape(N), prop)


# speedup vs baseline: 46.5238x; 2.3797x over previous
import functools

import jax
import jax.numpy as jnp
from jax import lax
from jax.experimental import pallas as pl
from jax.experimental.pallas import tpu as pltpu
from jax.experimental.pallas import tpu_sc as plsc

N = 16777216
SPARSITY = 0.01
SIGMOID_BIAS = 5.0
K = int(SPARSITY * N)

NC, NS, LANES = 2, 16, 16
NTILES = NC * NS
PER_TILE = N // NTILES
NBUF = 2
UNROLL = 8

B1, B2, B3 = 4096, 4096, 128

_mesh = plsc.VectorSubcoreMesh(core_axis_name="c", subcore_axis_name="s")
_sc_params = pltpu.CompilerParams(needs_layout_passes=False)


def _wid():
    return lax.axis_index("s") * NC + lax.axis_index("c")


def _zero_hist(hist_v, nwords):
    zeros = jnp.zeros((LANES,), jnp.int32)

    @plsc.parallel_loop(0, nwords // LANES, 1, unroll=UNROLL)
    def _(i):
        hist_v[pl.ds(i * LANES, LANES)] = zeros


_P1_BLK = 8192
_P1_NBLK = PER_TILE // _P1_BLK


@functools.partial(
    pl.kernel,
    out_type=[
        jax.ShapeDtypeStruct((N,), jnp.float32),
        jax.ShapeDtypeStruct((NTILES, LANES * B1), jnp.int32),
    ],
    mesh=_mesh,
    scratch_types=[
        pltpu.VMEM((NBUF, _P1_BLK), jnp.float32),
        pltpu.VMEM((NBUF, _P1_BLK), jnp.float32),
        pltpu.VMEM((NBUF, _P1_BLK), jnp.float32),
        pltpu.VMEM((LANES * B1,), jnp.int32),
        pltpu.SemaphoreType.DMA,
        pltpu.SemaphoreType.DMA,
        pltpu.SemaphoreType.DMA,
        pltpu.SemaphoreType.DMA,
        pltpu.SemaphoreType.DMA,
        pltpu.SemaphoreType.DMA,
    ],
    compiler_params=_sc_params,
)
def _p1(p_hbm, f_hbm, tv_hbm, hist_hbm, p_v, f_v, t_v, hist_v,
        sp0, sp1, sf0, sf1, ss0, ss1):
    wid = _wid()
    base = wid * PER_TILE
    sp = (sp0, sp1)
    sf = (sf0, sf1)
    ss = (ss0, ss1)
    for j in range(NBUF):
        off = base + j * _P1_BLK
        pltpu.async_copy(p_hbm.at[pl.ds(off, _P1_BLK)], p_v.at[j], sp[j])
        pltpu.async_copy(f_hbm.at[pl.ds(off, _P1_BLK)], f_v.at[j], sf[j])
    _zero_hist(hist_v, LANES * B1)
    lane = lax.iota(jnp.int32, LANES)
    ones = jnp.ones((LANES,), jnp.int32)

    def outer(g, _):
        for j in range(NBUF):
            blk = g * NBUF + j
            off = base + blk * _P1_BLK
            pltpu.make_async_copy(
                p_hbm.at[pl.ds(off, _P1_BLK)], p_v.at[j], sp[j]).wait()
            pltpu.make_async_copy(
                f_hbm.at[pl.ds(off, _P1_BLK)], f_v.at[j], sf[j]).wait()

            @pl.when(blk >= NBUF)
            def _wait_store():
                pltpu.make_async_copy(
                    t_v.at[j], tv_hbm.at[pl.ds(base, _P1_BLK)], ss[j]).wait()

            @plsc.parallel_loop(0, _P1_BLK // LANES, 1, unroll=UNROLL)
            def _vec(i):
                s = i * LANES
                t = f_v[j, pl.ds(s, LANES)] - p_v[j, pl.ds(s, LANES)]
                t_v[j, pl.ds(s, LANES)] = t
                bits = plsc.bitcast(jnp.abs(t), jnp.int32)
                bkt = lax.shift_right_logical(bits, 19)
                idx = lax.shift_left(bkt, 4) + lane
                plsc.addupdate_scatter(hist_v, [idx], ones)
            pltpu.async_copy(t_v.at[j], tv_hbm.at[pl.ds(off, _P1_BLK)], ss[j])

            @pl.when(blk + NBUF < _P1_NBLK)
            def _next_load():
                noff = base + (blk + NBUF) * _P1_BLK
                pltpu.async_copy(
                    p_hbm.at[pl.ds(noff, _P1_BLK)], p_v.at[j], sp[j])
                pltpu.async_copy(
                    f_hbm.at[pl.ds(noff, _P1_BLK)], f_v.at[j], sf[j])
        return 0

    lax.fori_loop(0, _P1_NBLK // NBUF, outer, 0)
    for j in range(NBUF):
        pltpu.make_async_copy(
            t_v.at[j], tv_hbm.at[pl.ds(base, _P1_BLK)], ss[j]).wait()
    pltpu.sync_copy(hist_v, hist_hbm.at[wid])


def _make_masked_hist(shift_match, shift_bucket, bmask, nbuckets, blk):
    nblk = PER_TILE // blk

    @functools.partial(
        pl.kernel,
        out_type=jax.ShapeDtypeStruct((NTILES, LANES * nbuckets), jnp.int32),
        mesh=_mesh,
        scratch_types=[
            pltpu.VMEM((NBUF, blk), jnp.float32),
            pltpu.VMEM((LANES,), jnp.int32),
            pltpu.VMEM((LANES * nbuckets,), jnp.int32),
            pltpu.SemaphoreType.DMA,
            pltpu.SemaphoreType.DMA,
        ],
        compiler_params=_sc_params,
    )
    def hist_pass(tv_hbm, sel_hbm, hist_hbm, tv_v, sel_v, hist_v, s0, s1):
        wid = _wid()
        base = wid * PER_TILE
        sems = (s0, s1)
        for j in range(NBUF):
            off = base + j * blk
            pltpu.async_copy(tv_hbm.at[pl.ds(off, blk)], tv_v.at[j], sems[j])
        pltpu.sync_copy(sel_hbm, sel_v)
        _zero_hist(hist_v, LANES * nbuckets)
        sel = sel_v[...]
        lane = lax.iota(jnp.int32, LANES)
        ones = jnp.ones((LANES,), jnp.int32)
        zeros = jnp.zeros((LANES,), jnp.int32)

        def outer(g, _):
            for j in range(NBUF):
                b = g * NBUF + j
                off = base + b * blk
                pltpu.make_async_copy(
                    tv_hbm.at[pl.ds(off, blk)], tv_v.at[j], sems[j]).wait()

                @plsc.parallel_loop(0, blk // LANES, 1, unroll=UNROLL)
                def _vec(i):
                    s = i * LANES
                    t = tv_v[j, pl.ds(s, LANES)]
                    bits = plsc.bitcast(jnp.abs(t), jnp.int32)
                    match = lax.shift_right_logical(bits, shift_match) == sel
                    bucket = jnp.bitwise_and(
                        lax.shift_right_logical(bits, shift_bucket), bmask)
                    idx = lax.shift_left(bucket, 4) + lane
                    plsc.addupdate_scatter(
                        hist_v, [idx], jnp.where(match, ones, zeros))

                @pl.when(b + NBUF < nblk)
                def _next_load():
                    noff = base + (b + NBUF) * blk
                    pltpu.async_copy(
                        tv_hbm.at[pl.ds(noff, blk)], tv_v.at[j], sems[j])
            return 0

        lax.fori_loop(0, nblk // NBUF, outer, 0)
        pltpu.sync_copy(hist_v, hist_hbm.at[wid])

    return hist_pass


_p2 = _make_masked_hist(19, 7, B2 - 1, B2, 16384)
_p3 = _make_masked_hist(7, 0, B3 - 1, B3, 32768)


def _select(hist, rank, nbuckets, nbits):
    def body(h_ref, r_ref, sel_ref, rn_ref):
        h = jnp.sum(h_ref[...], axis=0, keepdims=True)
        col = lax.shift_right_logical(
            lax.broadcasted_iota(jnp.int32, h.shape, 1), 4)
        k = r_ref[0, 0]

        def step(_, lohi):
            lo, hi = lohi
            mid = (lo + hi) // 2
            s = jnp.sum(jnp.where(col >= mid, h, 0))
            ok = s >= k
            return jnp.where(ok, mid, lo), jnp.where(ok, hi, mid)

        lo, _hi = lax.fori_loop(
            0, nbits, step, (jnp.int32(0), jnp.int32(nbuckets)))
        above = jnp.sum(jnp.where(col >= lo + 1, h, 0))
        sel_ref[0, 0] = lo
        rn_ref[0, 0] = k - above

    return pl.pallas_call(
        body,
        in_specs=[
            pl.BlockSpec(memory_space=pltpu.VMEM),
            pl.BlockSpec(memory_space=pltpu.SMEM),
        ],
        out_specs=[
            pl.BlockSpec(memory_space=pltpu.SMEM),
            pl.BlockSpec(memory_space=pltpu.SMEM),
        ],
        out_shape=[
            jax.ShapeDtypeStruct((1, 1), jnp.int32),
            jax.ShapeDtypeStruct((1, 1), jnp.int32),
        ],
    )(hist, rank)


_SIG_HI = 0.9933071490757153
_SIG_LO = 0.0066928509242848554

_EW_COLS = 1024
_EW_ROWS = N // _EW_COLS
_EW_GRID = 32
_EW_RB = _EW_ROWS // _EW_GRID


def _ew_body(thr_ref, tv_ref, mask_ref, delta_ref):
    thr = thr_ref[0, 0]
    t = tv_ref[...]
    above = jnp.abs(t) > thr
    mask_ref[...] = jnp.where(above, SIGMOID_BIAS, -SIGMOID_BIAS).astype(jnp.float32)
    delta_ref[...] = t * jnp.where(above, _SIG_HI, _SIG_LO).astype(jnp.float32)


def _elementwise(tv2d, thr):
    return pl.pallas_call(
        _ew_body,
        grid=(_EW_GRID,),
        in_specs=[
            pl.BlockSpec(memory_space=pltpu.SMEM),
            pl.BlockSpec((_EW_RB, _EW_COLS), lambda i: (i, 0)),
        ],
        out_specs=[
            pl.BlockSpec((_EW_RB, _EW_COLS), lambda i: (i, 0)),
            pl.BlockSpec((_EW_RB, _EW_COLS), lambda i: (i, 0)),
        ],
        out_shape=[
            jax.ShapeDtypeStruct((_EW_ROWS, _EW_COLS), jnp.float32),
            jax.ShapeDtypeStruct((_EW_ROWS, _EW_COLS), jnp.float32),
        ],
    )(thr, tv2d)


def kernel(pretrained, finetuned):
    tv, h1 = _p1(pretrained, finetuned)

    k0 = jnp.full((1, 1), K, jnp.int32)
    j1, r2 = _select(h1, k0, B1, 12)
    sel1 = jnp.full((LANES,), j1[0, 0], jnp.int32)

    h2 = _p2(tv, sel1)
    j2, r3 = _select(h2, r2, B2, 12)
    sel12 = jnp.full((LANES,), (j1[0, 0] << 12) | j2[0, 0], jnp.int32)

    h3 = _p3(tv, sel12)
    j3, r4 = _select(h3, r3, B3, 7)

    bits = (j1[0, 0] << 19) | (j2[0, 0] << 7) | j3[0, 0]
    thr = lax.bitcast_convert_type(bits, jnp.float32)

    mask2d, delta2d = _elementwise(tv.reshape(_EW_ROWS, _EW_COLS),
                                   jnp.full((1, 1), thr, jnp.float32))

    cnt_above = K - r4[0, 0]
    prop = (5 * (2 * cnt_above - N)).astype(jnp.float32) / jnp.float32(N)

    return (delta2d.reshape(N), mask2d.reshape(N), prop)


# speedup vs baseline: 54.0109x; 1.1609x over previous
import functools

import jax
import jax.numpy as jnp
from jax import lax
from jax.experimental import pallas as pl
from jax.experimental.pallas import tpu as pltpu
from jax.experimental.pallas import tpu_sc as plsc

N = 16777216
SPARSITY = 0.01
SIGMOID_BIAS = 5.0
K = int(SPARSITY * N)

NC, NS, LANES = 2, 16, 16
NTILES = NC * NS
PER_TILE = N // NTILES
NBUF = 2
UNROLL = 8

B1, B2, B3 = 4096, 4096, 128

_mesh = plsc.VectorSubcoreMesh(core_axis_name="c", subcore_axis_name="s")
_sc_params = pltpu.CompilerParams(needs_layout_passes=False)


def _wid():
    return lax.axis_index("s") * NC + lax.axis_index("c")


def _zero_hist(hist_v, nwords):
    zeros = jnp.zeros((LANES,), jnp.int32)

    @plsc.parallel_loop(0, nwords // LANES, 1, unroll=UNROLL)
    def _(i):
        hist_v[pl.ds(i * LANES, LANES)] = zeros


_P1_BLK = 8192
_P1_NBLK = PER_TILE // _P1_BLK


@functools.partial(
    pl.kernel,
    out_type=[
        jax.ShapeDtypeStruct((N,), jnp.float32),
        jax.ShapeDtypeStruct((NTILES, LANES * B1), jnp.int32),
    ],
    mesh=_mesh,
    scratch_types=[
        pltpu.VMEM((NBUF, _P1_BLK), jnp.float32),
        pltpu.VMEM((NBUF, _P1_BLK), jnp.float32),
        pltpu.VMEM((NBUF, _P1_BLK), jnp.float32),
        pltpu.VMEM((LANES * B1,), jnp.int32),
        pltpu.SemaphoreType.DMA,
        pltpu.SemaphoreType.DMA,
        pltpu.SemaphoreType.DMA,
        pltpu.SemaphoreType.DMA,
        pltpu.SemaphoreType.DMA,
        pltpu.SemaphoreType.DMA,
    ],
    compiler_params=_sc_params,
)
def _p1(p_hbm, f_hbm, tv_hbm, hist_hbm, p_v, f_v, t_v, hist_v,
        sp0, sp1, sf0, sf1, ss0, ss1):
    wid = _wid()
    base = wid * PER_TILE
    sp = (sp0, sp1)
    sf = (sf0, sf1)
    ss = (ss0, ss1)
    for j in range(NBUF):
        off = base + j * _P1_BLK
        pltpu.async_copy(p_hbm.at[pl.ds(off, _P1_BLK)], p_v.at[j], sp[j])
        pltpu.async_copy(f_hbm.at[pl.ds(off, _P1_BLK)], f_v.at[j], sf[j])
    _zero_hist(hist_v, LANES * B1)
    lane = lax.iota(jnp.int32, LANES)
    ones = jnp.ones((LANES,), jnp.int32)

    def outer(g, _):
        for j in range(NBUF):
            blk = g * NBUF + j
            off = base + blk * _P1_BLK
            pltpu.make_async_copy(
                p_hbm.at[pl.ds(off, _P1_BLK)], p_v.at[j], sp[j]).wait()
            pltpu.make_async_copy(
                f_hbm.at[pl.ds(off, _P1_BLK)], f_v.at[j], sf[j]).wait()

            @pl.when(blk >= NBUF)
            def _wait_store():
                pltpu.make_async_copy(
                    t_v.at[j], tv_hbm.at[pl.ds(base, _P1_BLK)], ss[j]).wait()

            @plsc.parallel_loop(0, _P1_BLK // LANES, 1, unroll=UNROLL)
            def _vec(i):
                s = i * LANES
                t = f_v[j, pl.ds(s, LANES)] - p_v[j, pl.ds(s, LANES)]
                t_v[j, pl.ds(s, LANES)] = t
                bits = plsc.bitcast(jnp.abs(t), jnp.int32)
                bkt = lax.shift_right_logical(bits, 19)
                idx = lax.shift_left(bkt, 4) + lane
                plsc.addupdate_scatter(hist_v, [idx], ones)
            pltpu.async_copy(t_v.at[j], tv_hbm.at[pl.ds(off, _P1_BLK)], ss[j])

            @pl.when(blk + NBUF < _P1_NBLK)
            def _next_load():
                noff = base + (blk + NBUF) * _P1_BLK
                pltpu.async_copy(
                    p_hbm.at[pl.ds(noff, _P1_BLK)], p_v.at[j], sp[j])
                pltpu.async_copy(
                    f_hbm.at[pl.ds(noff, _P1_BLK)], f_v.at[j], sf[j])
        return 0

    lax.fori_loop(0, _P1_NBLK // NBUF, outer, 0)
    for j in range(NBUF):
        pltpu.make_async_copy(
            t_v.at[j], tv_hbm.at[pl.ds(base, _P1_BLK)], ss[j]).wait()
    pltpu.sync_copy(hist_v, hist_hbm.at[wid])


def _make_masked_hist(shift_match, shift_bucket, bmask, nbuckets, blk):
    nblk = PER_TILE // blk

    @functools.partial(
        pl.kernel,
        out_type=jax.ShapeDtypeStruct((NTILES, LANES * nbuckets), jnp.int32),
        mesh=_mesh,
        scratch_types=[
            pltpu.VMEM((NBUF, blk), jnp.float32),
            pltpu.VMEM((LANES,), jnp.int32),
            pltpu.VMEM((LANES * nbuckets,), jnp.int32),
            pltpu.SemaphoreType.DMA,
            pltpu.SemaphoreType.DMA,
        ],
        compiler_params=_sc_params,
    )
    def hist_pass(tv_hbm, sel_hbm, hist_hbm, tv_v, sel_v, hist_v, s0, s1):
        wid = _wid()
        base = wid * PER_TILE
        sems = (s0, s1)
        for j in range(NBUF):
            off = base + j * blk
            pltpu.async_copy(tv_hbm.at[pl.ds(off, blk)], tv_v.at[j], sems[j])
        pltpu.sync_copy(sel_hbm, sel_v)
        _zero_hist(hist_v, LANES * nbuckets)
        sel = sel_v[...]
        lane = lax.iota(jnp.int32, LANES)
        ones = jnp.ones((LANES,), jnp.int32)
        zeros = jnp.zeros((LANES,), jnp.int32)

        def outer(g, _):
            for j in range(NBUF):
                b = g * NBUF + j
                off = base + b * blk
                pltpu.make_async_copy(
                    tv_hbm.at[pl.ds(off, blk)], tv_v.at[j], sems[j]).wait()

                @plsc.parallel_loop(0, blk // LANES, 1, unroll=UNROLL)
                def _vec(i):
                    s = i * LANES
                    t = tv_v[j, pl.ds(s, LANES)]
                    bits = plsc.bitcast(jnp.abs(t), jnp.int32)
                    match = lax.shift_right_logical(bits, shift_match) == sel
                    bucket = jnp.bitwise_and(
                        lax.shift_right_logical(bits, shift_bucket), bmask)
                    idx = lax.shift_left(bucket, 4) + lane
                    plsc.addupdate_scatter(
                        hist_v, [idx], jnp.where(match, ones, zeros))

                @pl.when(b + NBUF < nblk)
                def _next_load():
                    noff = base + (b + NBUF) * blk
                    pltpu.async_copy(
                        tv_hbm.at[pl.ds(noff, blk)], tv_v.at[j], sems[j])
            return 0

        lax.fori_loop(0, nblk // NBUF, outer, 0)
        pltpu.sync_copy(hist_v, hist_hbm.at[wid])

    return hist_pass


_p2 = _make_masked_hist(19, 7, B2 - 1, B2, 16384)
_p3 = _make_masked_hist(7, 0, B3 - 1, B3, 32768)


def _select(hist, rank, nbuckets, nbits):
    def body(h_ref, r_ref, sel_ref, rn_ref):
        h = jnp.sum(h_ref[...], axis=0, keepdims=True)
        col = lax.shift_right_logical(
            lax.broadcasted_iota(jnp.int32, h.shape, 1), 4)
        k = r_ref[0, 0]

        def step(_, lohi):
            lo, hi = lohi
            mid = (lo + hi) // 2
            s = jnp.sum(jnp.where(col >= mid, h, 0))
            ok = s >= k
            return jnp.where(ok, mid, lo), jnp.where(ok, hi, mid)

        lo, _hi = lax.fori_loop(
            0, nbits, step, (jnp.int32(0), jnp.int32(nbuckets)))
        above = jnp.sum(jnp.where(col >= lo + 1, h, 0))
        sel_ref[0, 0] = lo
        rn_ref[0, 0] = k - above

    return pl.pallas_call(
        body,
        in_specs=[
            pl.BlockSpec(memory_space=pltpu.VMEM),
            pl.BlockSpec(memory_space=pltpu.SMEM),
        ],
        out_specs=[
            pl.BlockSpec(memory_space=pltpu.SMEM),
            pl.BlockSpec(memory_space=pltpu.SMEM),
        ],
        out_shape=[
            jax.ShapeDtypeStruct((1, 1), jnp.int32),
            jax.ShapeDtypeStruct((1, 1), jnp.int32),
        ],
    )(hist, rank)


_SIG_HI = 0.9933071490757153
_SIG_LO = 0.0066928509242848554

_P4_BLK = 16384
_P4_NBLK = PER_TILE // _P4_BLK


@functools.partial(
    pl.kernel,
    out_type=[
        jax.ShapeDtypeStruct((N,), jnp.float32),
        jax.ShapeDtypeStruct((N,), jnp.float32),
    ],
    mesh=_mesh,
    scratch_types=[
        pltpu.VMEM((NBUF, _P4_BLK), jnp.float32),
        pltpu.VMEM((NBUF, _P4_BLK), jnp.float32),
        pltpu.VMEM((NBUF, _P4_BLK), jnp.float32),
        pltpu.VMEM((LANES,), jnp.float32),
        pltpu.SemaphoreType.DMA,
        pltpu.SemaphoreType.DMA,
        pltpu.SemaphoreType.DMA,
        pltpu.SemaphoreType.DMA,
        pltpu.SemaphoreType.DMA,
        pltpu.SemaphoreType.DMA,
    ],
    compiler_params=_sc_params,
)
def _p4(tv_hbm, thr_hbm, mask_hbm, delta_hbm, tv_v, m_v, d_v, thr_v,
        sl0, sl1, sm0, sm1, sd0, sd1):
    wid = _wid()
    base = wid * PER_TILE
    sl = (sl0, sl1)
    sm = (sm0, sm1)
    sd = (sd0, sd1)
    for j in range(NBUF):
        off = base + j * _P4_BLK
        pltpu.async_copy(tv_hbm.at[pl.ds(off, _P4_BLK)], tv_v.at[j], sl[j])
    pltpu.sync_copy(thr_hbm, thr_v)
    thr = thr_v[...]
    hi = jnp.full((LANES,), SIGMOID_BIAS, jnp.float32)
    lo = jnp.full((LANES,), -SIGMOID_BIAS, jnp.float32)
    shi = jnp.full((LANES,), _SIG_HI, jnp.float32)
    slo = jnp.full((LANES,), _SIG_LO, jnp.float32)

    def outer(g, _):
        for j in range(NBUF):
            blk = g * NBUF + j
            off = base + blk * _P4_BLK
            pltpu.make_async_copy(
                tv_hbm.at[pl.ds(off, _P4_BLK)], tv_v.at[j], sl[j]).wait()

            @pl.when(blk >= NBUF)
            def _wait_stores():
                pltpu.make_async_copy(
                    m_v.at[j], mask_hbm.at[pl.ds(base, _P4_BLK)], sm[j]).wait()
                pltpu.make_async_copy(
                    d_v.at[j], delta_hbm.at[pl.ds(base, _P4_BLK)], sd[j]).wait()

            @plsc.parallel_loop(0, _P4_BLK // LANES, 1, unroll=UNROLL)
            def _vec(i):
                s = i * LANES
                t = tv_v[j, pl.ds(s, LANES)]
                above = jnp.abs(t) > thr
                m_v[j, pl.ds(s, LANES)] = jnp.where(above, hi, lo)
                d_v[j, pl.ds(s, LANES)] = t * jnp.where(above, shi, slo)

            pltpu.async_copy(m_v.at[j], mask_hbm.at[pl.ds(off, _P4_BLK)], sm[j])
            pltpu.async_copy(d_v.at[j], delta_hbm.at[pl.ds(off, _P4_BLK)], sd[j])

            @pl.when(blk + NBUF < _P4_NBLK)
            def _next_load():
                noff = base + (blk + NBUF) * _P4_BLK
                pltpu.async_copy(
                    tv_hbm.at[pl.ds(noff, _P4_BLK)], tv_v.at[j], sl[j])
        return 0

    lax.fori_loop(0, _P4_NBLK // NBUF, outer, 0)
    for j in range(NBUF):
        pltpu.make_async_copy(
            m_v.at[j], mask_hbm.at[pl.ds(base, _P4_BLK)], sm[j]).wait()
        pltpu.make_async_copy(
            d_v.at[j], delta_hbm.at[pl.ds(base, _P4_BLK)], sd[j]).wait()


def kernel(pretrained, finetuned):
    tv, h1 = _p1(pretrained, finetuned)

    k0 = jnp.full((1, 1), K, jnp.int32)
    j1, r2 = _select(h1, k0, B1, 12)
    sel1 = jnp.full((LANES,), j1[0, 0], jnp.int32)

    h2 = _p2(tv, sel1)
    j2, r3 = _select(h2, r2, B2, 12)
    sel12 = jnp.full((LANES,), (j1[0, 0] << 12) | j2[0, 0], jnp.int32)

    h3 = _p3(tv, sel12)
    j3, r4 = _select(h3, r3, B3, 7)

    bits = (j1[0, 0] << 19) | (j2[0, 0] << 7) | j3[0, 0]
    thr = lax.bitcast_convert_type(bits, jnp.float32)

    mask, delta = _p4(tv, jnp.full((LANES,), thr, jnp.float32))

    cnt_above = K - r4[0, 0]
    prop = (5 * (2 * cnt_above - N)).astype(jnp.float32) / jnp.float32(N)

    return (delta, mask, prop)


# speedup vs baseline: 56.2926x; 1.0422x over previous
import functools

import jax
import jax.numpy as jnp
from jax import lax
from jax.experimental import pallas as pl
from jax.experimental.pallas import tpu as pltpu
from jax.experimental.pallas import tpu_sc as plsc

N = 16777216
SPARSITY = 0.01
SIGMOID_BIAS = 5.0
K = int(SPARSITY * N)

NC, NS, LANES = 2, 16, 16
NTILES = NC * NS
PER_TILE = N // NTILES
NBUF = 2
UNROLL = 8

B1, B2, B3 = 4096, 4096, 128

_mesh = plsc.VectorSubcoreMesh(core_axis_name="c", subcore_axis_name="s")
_sc_params = pltpu.CompilerParams(needs_layout_passes=False)


def _wid():
    return lax.axis_index("s") * NC + lax.axis_index("c")


def _zero_hist(hist_v, nwords):
    zeros = jnp.zeros((LANES,), jnp.int32)

    @plsc.parallel_loop(0, nwords // LANES, 1, unroll=UNROLL)
    def _(i):
        hist_v[pl.ds(i * LANES, LANES)] = zeros


_P1_BLK = 8192
_P1_NBLK = PER_TILE // _P1_BLK


@functools.partial(
    pl.kernel,
    out_type=[
        jax.ShapeDtypeStruct((N,), jnp.float32),
        jax.ShapeDtypeStruct((NTILES, LANES * B1), jnp.int32),
    ],
    mesh=_mesh,
    scratch_types=[
        pltpu.VMEM((NBUF, _P1_BLK), jnp.float32),
        pltpu.VMEM((NBUF, _P1_BLK), jnp.float32),
        pltpu.VMEM((NBUF, _P1_BLK), jnp.float32),
        pltpu.VMEM((LANES * B1,), jnp.int32),
        pltpu.SemaphoreType.DMA,
        pltpu.SemaphoreType.DMA,
        pltpu.SemaphoreType.DMA,
        pltpu.SemaphoreType.DMA,
        pltpu.SemaphoreType.DMA,
        pltpu.SemaphoreType.DMA,
    ],
    compiler_params=_sc_params,
)
def _p1(p_hbm, f_hbm, tv_hbm, hist_hbm, p_v, f_v, t_v, hist_v,
        sp0, sp1, sf0, sf1, ss0, ss1):
    wid = _wid()
    base = wid * PER_TILE
    sp = (sp0, sp1)
    sf = (sf0, sf1)
    ss = (ss0, ss1)
    for j in range(NBUF):
        off = base + j * _P1_BLK
        pltpu.async_copy(p_hbm.at[pl.ds(off, _P1_BLK)], p_v.at[j], sp[j])
        pltpu.async_copy(f_hbm.at[pl.ds(off, _P1_BLK)], f_v.at[j], sf[j])
    _zero_hist(hist_v, LANES * B1)
    lane = lax.iota(jnp.int32, LANES)
    ones = jnp.ones((LANES,), jnp.int32)

    def outer(g, _):
        for j in range(NBUF):
            blk = g * NBUF + j
            off = base + blk * _P1_BLK
            pltpu.make_async_copy(
                p_hbm.at[pl.ds(off, _P1_BLK)], p_v.at[j], sp[j]).wait()
            pltpu.make_async_copy(
                f_hbm.at[pl.ds(off, _P1_BLK)], f_v.at[j], sf[j]).wait()

            @pl.when(blk >= NBUF)
            def _wait_store():
                pltpu.make_async_copy(
                    t_v.at[j], tv_hbm.at[pl.ds(base, _P1_BLK)], ss[j]).wait()

            @plsc.parallel_loop(0, _P1_BLK // LANES, 1, unroll=UNROLL)
            def _vec(i):
                s = i * LANES
                t = f_v[j, pl.ds(s, LANES)] - p_v[j, pl.ds(s, LANES)]
                t_v[j, pl.ds(s, LANES)] = t
                bits = plsc.bitcast(jnp.abs(t), jnp.int32)
                bkt = lax.shift_right_logical(bits, 19)
                idx = lax.shift_left(bkt, 4) + lane
                plsc.addupdate_scatter(hist_v, [idx], ones)
            pltpu.async_copy(t_v.at[j], tv_hbm.at[pl.ds(off, _P1_BLK)], ss[j])

            @pl.when(blk + NBUF < _P1_NBLK)
            def _next_load():
                noff = base + (blk + NBUF) * _P1_BLK
                pltpu.async_copy(
                    p_hbm.at[pl.ds(noff, _P1_BLK)], p_v.at[j], sp[j])
                pltpu.async_copy(
                    f_hbm.at[pl.ds(noff, _P1_BLK)], f_v.at[j], sf[j])
        return 0

    lax.fori_loop(0, _P1_NBLK // NBUF, outer, 0)
    for j in range(NBUF):
        pltpu.make_async_copy(
            t_v.at[j], tv_hbm.at[pl.ds(base, _P1_BLK)], ss[j]).wait()
    pltpu.sync_copy(hist_v, hist_hbm.at[wid])


def _make_masked_hist(shift_match, shift_bucket, bmask, nbuckets, blk):
    nblk = PER_TILE // blk

    @functools.partial(
        pl.kernel,
        out_type=jax.ShapeDtypeStruct((NTILES, LANES * nbuckets), jnp.int32),
        mesh=_mesh,
        scratch_types=[
            pltpu.VMEM((NBUF, blk), jnp.float32),
            pltpu.VMEM((LANES,), jnp.int32),
            pltpu.VMEM((LANES * nbuckets,), jnp.int32),
            pltpu.SemaphoreType.DMA,
            pltpu.SemaphoreType.DMA,
        ],
        compiler_params=_sc_params,
    )
    def hist_pass(tv_hbm, sel_hbm, hist_hbm, tv_v, sel_v, hist_v, s0, s1):
        wid = _wid()
        base = wid * PER_TILE
        sems = (s0, s1)
        for j in range(NBUF):
            off = base + j * blk
            pltpu.async_copy(tv_hbm.at[pl.ds(off, blk)], tv_v.at[j], sems[j])
        pltpu.sync_copy(sel_hbm, sel_v)
        _zero_hist(hist_v, LANES * nbuckets)
        sel = sel_v[...]
        lane = lax.iota(jnp.int32, LANES)
        ones = jnp.ones((LANES,), jnp.int32)

        def outer(g, _):
            for j in range(NBUF):
                b = g * NBUF + j
                off = base + b * blk
                pltpu.make_async_copy(
                    tv_hbm.at[pl.ds(off, blk)], tv_v.at[j], sems[j]).wait()

                @plsc.parallel_loop(0, blk // LANES, 1, unroll=UNROLL)
                def _vec(i):
                    s = i * LANES
                    t = tv_v[j, pl.ds(s, LANES)]
                    bits = plsc.bitcast(jnp.abs(t), jnp.int32)
                    match = lax.shift_right_logical(bits, shift_match) == sel
                    bucket = jnp.bitwise_and(
                        lax.shift_right_logical(bits, shift_bucket), bmask)
                    idx = lax.shift_left(bucket, 4) + lane
                    plsc.addupdate_scatter(hist_v, [idx], ones, mask=match)

                @pl.when(b + NBUF < nblk)
                def _next_load():
                    noff = base + (b + NBUF) * blk
                    pltpu.async_copy(
                        tv_hbm.at[pl.ds(noff, blk)], tv_v.at[j], sems[j])
            return 0

        lax.fori_loop(0, nblk // NBUF, outer, 0)
        pltpu.sync_copy(hist_v, hist_hbm.at[wid])

    return hist_pass


_p2 = _make_masked_hist(19, 7, B2 - 1, B2, 16384)
_p3 = _make_masked_hist(7, 0, B3 - 1, B3, 32768)


def _select(hist, rank, prev, nbuckets, nbits, last=False):
    def body(h_ref, r_ref, p_ref, selv_ref, sels_ref, rn_ref, *prop_ref):
        h = jnp.sum(h_ref[...], axis=0, keepdims=True)
        col = lax.shift_right_logical(
            lax.broadcasted_iota(jnp.int32, h.shape, 1), 4)
        k = r_ref[0, 0]

        def step(_, lohi):
            lo, hi = lohi
            mid = (lo + hi) // 2
            s = jnp.sum(jnp.where(col >= mid, h, 0))
            ok = s >= k
            return jnp.where(ok, mid, lo), jnp.where(ok, hi, mid)

        lo, _hi = lax.fori_loop(
            0, nbits, step, (jnp.int32(0), jnp.int32(nbuckets)))
        above = jnp.sum(jnp.where(col >= lo + 1, h, 0))
        combined = (p_ref[0, 0] << nbits) | lo
        rnext = k - above
        selv_ref[...] = jnp.full((LANES,), combined, jnp.int32)
        sels_ref[0, 0] = combined
        rn_ref[0, 0] = rnext
        if last:
            cnt_above = K - rnext
            prop_ref[0][0, 0] = (
                (5 * (2 * cnt_above - N)).astype(jnp.float32) / jnp.float32(N))

    out_specs = [
        pl.BlockSpec(memory_space=pltpu.VMEM),
        pl.BlockSpec(memory_space=pltpu.SMEM),
        pl.BlockSpec(memory_space=pltpu.SMEM),
    ]
    out_shape = [
        jax.ShapeDtypeStruct((LANES,), jnp.int32),
        jax.ShapeDtypeStruct((1, 1), jnp.int32),
        jax.ShapeDtypeStruct((1, 1), jnp.int32),
    ]
    if last:
        out_specs.append(pl.BlockSpec(memory_space=pltpu.SMEM))
        out_shape.append(jax.ShapeDtypeStruct((1, 1), jnp.float32))
    return pl.pallas_call(
        body,
        in_specs=[
            pl.BlockSpec(memory_space=pltpu.VMEM),
            pl.BlockSpec(memory_space=pltpu.SMEM),
            pl.BlockSpec(memory_space=pltpu.SMEM),
        ],
        out_specs=out_specs,
        out_shape=out_shape,
    )(hist, rank, prev)


_SIG_HI = 0.9933071490757153
_SIG_LO = 0.0066928509242848554

_P4_BLK = 16384
_P4_NBLK = PER_TILE // _P4_BLK


@functools.partial(
    pl.kernel,
    out_type=[
        jax.ShapeDtypeStruct((N,), jnp.float32),
        jax.ShapeDtypeStruct((N,), jnp.float32),
    ],
    mesh=_mesh,
    scratch_types=[
        pltpu.VMEM((NBUF, _P4_BLK), jnp.float32),
        pltpu.VMEM((NBUF, _P4_BLK), jnp.float32),
        pltpu.VMEM((NBUF, _P4_BLK), jnp.float32),
        pltpu.VMEM((LANES,), jnp.int32),
        pltpu.SemaphoreType.DMA,
        pltpu.SemaphoreType.DMA,
        pltpu.SemaphoreType.DMA,
        pltpu.SemaphoreType.DMA,
        pltpu.SemaphoreType.DMA,
        pltpu.SemaphoreType.DMA,
    ],
    compiler_params=_sc_params,
)
def _p4(tv_hbm, thr_hbm, mask_hbm, delta_hbm, tv_v, m_v, d_v, thr_v,
        sl0, sl1, sm0, sm1, sd0, sd1):
    wid = _wid()
    base = wid * PER_TILE
    sl = (sl0, sl1)
    sm = (sm0, sm1)
    sd = (sd0, sd1)
    for j in range(NBUF):
        off = base + j * _P4_BLK
        pltpu.async_copy(tv_hbm.at[pl.ds(off, _P4_BLK)], tv_v.at[j], sl[j])
    pltpu.sync_copy(thr_hbm, thr_v)
    thr = plsc.bitcast(thr_v[...], jnp.float32)
    hi = jnp.full((LANES,), SIGMOID_BIAS, jnp.float32)
    lo = jnp.full((LANES,), -SIGMOID_BIAS, jnp.float32)
    shi = jnp.full((LANES,), _SIG_HI, jnp.float32)
    slo = jnp.full((LANES,), _SIG_LO, jnp.float32)

    def outer(g, _):
        for j in range(NBUF):
            blk = g * NBUF + j
            off = base + blk * _P4_BLK
            pltpu.make_async_copy(
                tv_hbm.at[pl.ds(off, _P4_BLK)], tv_v.at[j], sl[j]).wait()

            @pl.when(blk >= NBUF)
            def _wait_stores():
                pltpu.make_async_copy(
                    m_v.at[j], mask_hbm.at[pl.ds(base, _P4_BLK)], sm[j]).wait()
                pltpu.make_async_copy(
                    d_v.at[j], delta_hbm.at[pl.ds(base, _P4_BLK)], sd[j]).wait()

            @plsc.parallel_loop(0, _P4_BLK // LANES, 1, unroll=UNROLL)
            def _vec(i):
                s = i * LANES
                t = tv_v[j, pl.ds(s, LANES)]
                above = jnp.abs(t) > thr
                m_v[j, pl.ds(s, LANES)] = jnp.where(above, hi, lo)
                d_v[j, pl.ds(s, LANES)] = t * jnp.where(above, shi, slo)

            pltpu.async_copy(m_v.at[j], mask_hbm.at[pl.ds(off, _P4_BLK)], sm[j])
            pltpu.async_copy(d_v.at[j], delta_hbm.at[pl.ds(off, _P4_BLK)], sd[j])

            @pl.when(blk + NBUF < _P4_NBLK)
            def _next_load():
                noff = base + (blk + NBUF) * _P4_BLK
                pltpu.async_copy(
                    tv_hbm.at[pl.ds(noff, _P4_BLK)], tv_v.at[j], sl[j])
        return 0

    lax.fori_loop(0, _P4_NBLK // NBUF, outer, 0)
    for j in range(NBUF):
        pltpu.make_async_copy(
            m_v.at[j], mask_hbm.at[pl.ds(base, _P4_BLK)], sm[j]).wait()
        pltpu.make_async_copy(
            d_v.at[j], delta_hbm.at[pl.ds(base, _P4_BLK)], sd[j]).wait()


def kernel(pretrained, finetuned):
    tv, h1 = _p1(pretrained, finetuned)

    k0 = jnp.full((1, 1), K, jnp.int32)
    zero = jnp.zeros((1, 1), jnp.int32)
    sel1v, j1s, r2 = _select(h1, k0, zero, B1, 12)
    h2 = _p2(tv, sel1v)
    sel12v, j12s, r3 = _select(h2, r2, j1s, B2, 12)
    h3 = _p3(tv, sel12v)
    thrv, _bits, _r4, prop11 = _select(h3, r3, j12s, B3, 7, last=True)

    mask, delta = _p4(tv, thrv)
    return (delta, mask, prop11[0, 0])


# speedup vs baseline: 58.1523x; 1.0330x over previous
import functools

import jax
import jax.numpy as jnp
from jax import lax
from jax.experimental import pallas as pl
from jax.experimental.pallas import tpu as pltpu
from jax.experimental.pallas import tpu_sc as plsc

N = 16777216
SPARSITY = 0.01
SIGMOID_BIAS = 5.0
K = int(SPARSITY * N)

NC, NS, LANES = 2, 16, 16
NTILES = NC * NS
PER_TILE = N // NTILES
NBUF = 2
UNROLL = 8

B1, B2, B3 = 2048, 2048, 512

_mesh = plsc.VectorSubcoreMesh(core_axis_name="c", subcore_axis_name="s")
_sc_params = pltpu.CompilerParams(needs_layout_passes=False)


def _wid():
    return lax.axis_index("s") * NC + lax.axis_index("c")


def _zero_hist(hist_v, nwords):
    zeros = jnp.zeros((LANES,), jnp.int32)

    @plsc.parallel_loop(0, nwords // LANES, 1, unroll=UNROLL)
    def _(i):
        hist_v[pl.ds(i * LANES, LANES)] = zeros


_P1_BLK = 4096
_P1_NBLK = PER_TILE // _P1_BLK
_P1_NBUF = 4


@functools.partial(
    pl.kernel,
    out_type=[
        jax.ShapeDtypeStruct((N,), jnp.float32),
        jax.ShapeDtypeStruct((NTILES, LANES * B1), jnp.int32),
    ],
    mesh=_mesh,
    scratch_types=[
        pltpu.VMEM((_P1_NBUF, _P1_BLK), jnp.float32),
        pltpu.VMEM((_P1_NBUF, _P1_BLK), jnp.float32),
        pltpu.VMEM((_P1_NBUF, _P1_BLK), jnp.float32),
        pltpu.VMEM((LANES * B1,), jnp.int32),
        pltpu.SemaphoreType.DMA,
        pltpu.SemaphoreType.DMA,
        pltpu.SemaphoreType.DMA,
        pltpu.SemaphoreType.DMA,
        pltpu.SemaphoreType.DMA,
        pltpu.SemaphoreType.DMA,
        pltpu.SemaphoreType.DMA,
        pltpu.SemaphoreType.DMA,
        pltpu.SemaphoreType.DMA,
        pltpu.SemaphoreType.DMA,
        pltpu.SemaphoreType.DMA,
        pltpu.SemaphoreType.DMA,
    ],
    compiler_params=_sc_params,
)
def _p1(p_hbm, f_hbm, tv_hbm, hist_hbm, p_v, f_v, t_v, hist_v,
        sp0, sp1, sp2, sp3, sf0, sf1, sf2, sf3, ss0, ss1, ss2, ss3):
    wid = _wid()
    base = wid * PER_TILE
    sp = (sp0, sp1, sp2, sp3)
    sf = (sf0, sf1, sf2, sf3)
    ss = (ss0, ss1, ss2, ss3)
    for j in range(_P1_NBUF):
        off = base + j * _P1_BLK
        pltpu.async_copy(p_hbm.at[pl.ds(off, _P1_BLK)], p_v.at[j], sp[j])
        pltpu.async_copy(f_hbm.at[pl.ds(off, _P1_BLK)], f_v.at[j], sf[j])
    _zero_hist(hist_v, LANES * B1)
    lane = lax.iota(jnp.int32, LANES)
    ones = jnp.ones((LANES,), jnp.int32)

    def outer(g, _):
        for j in range(_P1_NBUF):
            blk = g * _P1_NBUF + j
            off = base + blk * _P1_BLK
            pltpu.make_async_copy(
                p_hbm.at[pl.ds(off, _P1_BLK)], p_v.at[j], sp[j]).wait()
            pltpu.make_async_copy(
                f_hbm.at[pl.ds(off, _P1_BLK)], f_v.at[j], sf[j]).wait()

            @pl.when(blk >= _P1_NBUF)
            def _wait_store():
                pltpu.make_async_copy(
                    t_v.at[j], tv_hbm.at[pl.ds(base, _P1_BLK)], ss[j]).wait()

            @plsc.parallel_loop(0, _P1_BLK // LANES, 1, unroll=UNROLL)
            def _vec(i):
                s = i * LANES
                t = f_v[j, pl.ds(s, LANES)] - p_v[j, pl.ds(s, LANES)]
                t_v[j, pl.ds(s, LANES)] = t
                bits = plsc.bitcast(jnp.abs(t), jnp.int32)
                bkt = lax.shift_right_logical(bits, 20)
                idx = lax.shift_left(bkt, 4) + lane
                plsc.addupdate_scatter(hist_v, [idx], ones)
            pltpu.async_copy(t_v.at[j], tv_hbm.at[pl.ds(off, _P1_BLK)], ss[j])

            @pl.when(blk + _P1_NBUF < _P1_NBLK)
            def _next_load():
                noff = base + (blk + _P1_NBUF) * _P1_BLK
                pltpu.async_copy(
                    p_hbm.at[pl.ds(noff, _P1_BLK)], p_v.at[j], sp[j])
                pltpu.async_copy(
                    f_hbm.at[pl.ds(noff, _P1_BLK)], f_v.at[j], sf[j])
        return 0

    lax.fori_loop(0, _P1_NBLK // _P1_NBUF, outer, 0)
    for j in range(_P1_NBUF):
        pltpu.make_async_copy(
            t_v.at[j], tv_hbm.at[pl.ds(base, _P1_BLK)], ss[j]).wait()
    pltpu.sync_copy(hist_v, hist_hbm.at[wid])


def _make_masked_hist(shift_match, shift_bucket, bmask, nbuckets, blk):
    nblk = PER_TILE // blk

    @functools.partial(
        pl.kernel,
        out_type=jax.ShapeDtypeStruct((NTILES, LANES * nbuckets), jnp.int32),
        mesh=_mesh,
        scratch_types=[
            pltpu.VMEM((NBUF, blk), jnp.float32),
            pltpu.VMEM((LANES,), jnp.int32),
            pltpu.VMEM((LANES * nbuckets,), jnp.int32),
            pltpu.SemaphoreType.DMA,
            pltpu.SemaphoreType.DMA,
        ],
        compiler_params=_sc_params,
    )
    def hist_pass(tv_hbm, sel_hbm, hist_hbm, tv_v, sel_v, hist_v, s0, s1):
        wid = _wid()
        base = wid * PER_TILE
        sems = (s0, s1)
        for j in range(NBUF):
            off = base + j * blk
            pltpu.async_copy(tv_hbm.at[pl.ds(off, blk)], tv_v.at[j], sems[j])
        pltpu.sync_copy(sel_hbm, sel_v)
        _zero_hist(hist_v, LANES * nbuckets)
        sel = sel_v[...]
        lane = lax.iota(jnp.int32, LANES)
        ones = jnp.ones((LANES,), jnp.int32)

        def outer(g, _):
            for j in range(NBUF):
                b = g * NBUF + j
                off = base + b * blk
                pltpu.make_async_copy(
                    tv_hbm.at[pl.ds(off, blk)], tv_v.at[j], sems[j]).wait()

                @plsc.parallel_loop(0, blk // LANES, 1, unroll=UNROLL)
                def _vec(i):
                    s = i * LANES
                    t = tv_v[j, pl.ds(s, LANES)]
                    bits = plsc.bitcast(jnp.abs(t), jnp.int32)
                    match = lax.shift_right_logical(bits, shift_match) == sel
                    bucket = jnp.bitwise_and(
                        lax.shift_right_logical(bits, shift_bucket), bmask)
                    idx = lax.shift_left(bucket, 4) + lane
                    plsc.addupdate_scatter(hist_v, [idx], ones, mask=match)

                @pl.when(b + NBUF < nblk)
                def _next_load():
                    noff = base + (b + NBUF) * blk
                    pltpu.async_copy(
                        tv_hbm.at[pl.ds(noff, blk)], tv_v.at[j], sems[j])
            return 0

        lax.fori_loop(0, nblk // NBUF, outer, 0)
        pltpu.sync_copy(hist_v, hist_hbm.at[wid])

    return hist_pass


_p2 = _make_masked_hist(20, 9, B2 - 1, B2, 16384)
_p3 = _make_masked_hist(9, 0, B3 - 1, B3, 32768)


def _select(hist, rank, prev, nbuckets, nbits, last=False):
    def body(h_ref, r_ref, p_ref, selv_ref, sels_ref, rn_ref, *prop_ref):
        h = jnp.sum(h_ref[...], axis=0, keepdims=True)
        col = lax.shift_right_logical(
            lax.broadcasted_iota(jnp.int32, h.shape, 1), 4)
        k = r_ref[0, 0]

        def step(_, lohi):
            lo, hi = lohi
            mid = (lo + hi) // 2
            s = jnp.sum(jnp.where(col >= mid, h, 0))
            ok = s >= k
            return jnp.where(ok, mid, lo), jnp.where(ok, hi, mid)

        lo, _hi = lax.fori_loop(
            0, nbits, step, (jnp.int32(0), jnp.int32(nbuckets)))
        above = jnp.sum(jnp.where(col >= lo + 1, h, 0))
        combined = (p_ref[0, 0] << nbits) | lo
        rnext = k - above
        selv_ref[...] = jnp.full((LANES,), combined, jnp.int32)
        sels_ref[0, 0] = combined
        rn_ref[0, 0] = rnext
        if last:
            cnt_above = K - rnext
            prop_ref[0][0, 0] = (
                (5 * (2 * cnt_above - N)).astype(jnp.float32) / jnp.float32(N))

    out_specs = [
        pl.BlockSpec(memory_space=pltpu.VMEM),
        pl.BlockSpec(memory_space=pltpu.SMEM),
        pl.BlockSpec(memory_space=pltpu.SMEM),
    ]
    out_shape = [
        jax.ShapeDtypeStruct((LANES,), jnp.int32),
        jax.ShapeDtypeStruct((1, 1), jnp.int32),
        jax.ShapeDtypeStruct((1, 1), jnp.int32),
    ]
    if last:
        out_specs.append(pl.BlockSpec(memory_space=pltpu.SMEM))
        out_shape.append(jax.ShapeDtypeStruct((1, 1), jnp.float32))
    return pl.pallas_call(
        body,
        in_specs=[
            pl.BlockSpec(memory_space=pltpu.VMEM),
            pl.BlockSpec(memory_space=pltpu.SMEM),
            pl.BlockSpec(memory_space=pltpu.SMEM),
        ],
        out_specs=out_specs,
        out_shape=out_shape,
    )(hist, rank, prev)


_SIG_HI = 0.9933071490757153
_SIG_LO = 0.0066928509242848554

_P4_BLK = 16384
_P4_NBLK = PER_TILE // _P4_BLK


@functools.partial(
    pl.kernel,
    out_type=[
        jax.ShapeDtypeStruct((N,), jnp.float32),
        jax.ShapeDtypeStruct((N,), jnp.float32),
    ],
    mesh=_mesh,
    scratch_types=[
        pltpu.VMEM((NBUF, _P4_BLK), jnp.float32),
        pltpu.VMEM((NBUF, _P4_BLK), jnp.float32),
        pltpu.VMEM((NBUF, _P4_BLK), jnp.float32),
        pltpu.VMEM((LANES,), jnp.int32),
        pltpu.SemaphoreType.DMA,
        pltpu.SemaphoreType.DMA,
        pltpu.SemaphoreType.DMA,
        pltpu.SemaphoreType.DMA,
        pltpu.SemaphoreType.DMA,
        pltpu.SemaphoreType.DMA,
    ],
    compiler_params=_sc_params,
)
def _p4(tv_hbm, thr_hbm, mask_hbm, delta_hbm, tv_v, m_v, d_v, thr_v,
        sl0, sl1, sm0, sm1, sd0, sd1):
    wid = _wid()
    base = wid * PER_TILE
    sl = (sl0, sl1)
    sm = (sm0, sm1)
    sd = (sd0, sd1)
    for j in range(NBUF):
        off = base + j * _P4_BLK
        pltpu.async_copy(tv_hbm.at[pl.ds(off, _P4_BLK)], tv_v.at[j], sl[j])
    pltpu.sync_copy(thr_hbm, thr_v)
    thr = plsc.bitcast(thr_v[...], jnp.float32)
    hi = jnp.full((LANES,), SIGMOID_BIAS, jnp.float32)
    lo = jnp.full((LANES,), -SIGMOID_BIAS, jnp.float32)
    shi = jnp.full((LANES,), _SIG_HI, jnp.float32)
    slo = jnp.full((LANES,), _SIG_LO, jnp.float32)

    def outer(g, _):
        for j in range(NBUF):
            blk = g * NBUF + j
            off = base + blk * _P4_BLK
            pltpu.make_async_copy(
                tv_hbm.at[pl.ds(off, _P4_BLK)], tv_v.at[j], sl[j]).wait()

            @pl.when(blk >= NBUF)
            def _wait_stores():
                pltpu.make_async_copy(
                    m_v.at[j], mask_hbm.at[pl.ds(base, _P4_BLK)], sm[j]).wait()
                pltpu.make_async_copy(
                    d_v.at[j], delta_hbm.at[pl.ds(base, _P4_BLK)], sd[j]).wait()

            @plsc.parallel_loop(0, _P4_BLK // LANES, 1, unroll=UNROLL)
            def _vec(i):
                s = i * LANES
                t = tv_v[j, pl.ds(s, LANES)]
                above = jnp.abs(t) > thr
                m_v[j, pl.ds(s, LANES)] = jnp.where(above, hi, lo)
                d_v[j, pl.ds(s, LANES)] = t * jnp.where(above, shi, slo)

            pltpu.async_copy(m_v.at[j], mask_hbm.at[pl.ds(off, _P4_BLK)], sm[j])
            pltpu.async_copy(d_v.at[j], delta_hbm.at[pl.ds(off, _P4_BLK)], sd[j])

            @pl.when(blk + NBUF < _P4_NBLK)
            def _next_load():
                noff = base + (blk + NBUF) * _P4_BLK
                pltpu.async_copy(
                    tv_hbm.at[pl.ds(noff, _P4_BLK)], tv_v.at[j], sl[j])
        return 0

    lax.fori_loop(0, _P4_NBLK // NBUF, outer, 0)
    for j in range(NBUF):
        pltpu.make_async_copy(
            m_v.at[j], mask_hbm.at[pl.ds(base, _P4_BLK)], sm[j]).wait()
        pltpu.make_async_copy(
            d_v.at[j], delta_hbm.at[pl.ds(base, _P4_BLK)], sd[j]).wait()


def kernel(pretrained, finetuned):
    tv, h1 = _p1(pretrained, finetuned)

    k0 = jnp.full((1, 1), K, jnp.int32)
    zero = jnp.zeros((1, 1), jnp.int32)
    sel1v, j1s, r2 = _select(h1, k0, zero, B1, 11)
    h2 = _p2(tv, sel1v)
    sel12v, j12s, r3 = _select(h2, r2, j1s, B2, 11)
    h3 = _p3(tv, sel12v)
    thrv, _bits, _r4, prop11 = _select(h3, r3, j12s, B3, 9, last=True)

    mask, delta = _p4(tv, thrv)
    return (delta, mask, prop11[0, 0])
